# speedup vs baseline: 2.5253x; 2.5253x over previous
import functools

import jax
import jax.numpy as jnp
from jax import lax
from jax.experimental import pallas as pl
from jax.experimental.pallas import tpu as pltpu
from jax.experimental.pallas import tpu_sc as plsc

N_NODES = 10000
N_PAIRS = 160000
D_FEAT = 256

NC = 2
NS = 16
NW = NC * NS

PAIRS_PER_W = N_PAIRS // NW
C = 40
NCHUNK = PAIRS_PER_W // C

ROWS_PER_W = (N_NODES // (8 * NW)) * 8
COPY_TAIL = N_NODES - ROWS_PER_W * NW
NVEC = D_FEAT // 16


@functools.partial(
    pl.kernel,
    mesh=plsc.VectorSubcoreMesh(core_axis_name="c", subcore_axis_name="s"),
    out_type=jax.ShapeDtypeStruct((N_NODES + N_PAIRS, D_FEAT), jnp.float32),
    scratch_types=[
        pltpu.VMEM((2 * C,), jnp.int32),
        pltpu.VMEM((2 * C, D_FEAT), jnp.float32),
        pltpu.VMEM((C, D_FEAT), jnp.float32),
        pltpu.VMEM((ROWS_PER_W, D_FEAT), jnp.float32),
        pltpu.SemaphoreType.DMA,
    ],
)
def _pool_kernel(x_hbm, idx_hbm, out_hbm, idx_v, rows_v, acc_v, copy_v, sem):
    wid = lax.axis_index("s") * NC + lax.axis_index("c")

    rbase = wid * ROWS_PER_W
    pltpu.sync_copy(x_hbm.at[pl.ds(rbase, ROWS_PER_W)], copy_v)
    pltpu.sync_copy(copy_v, out_hbm.at[pl.ds(rbase, ROWS_PER_W)])

    @pl.when(wid == 0)
    def _copy_tail():
        tbase = NW * ROWS_PER_W
        pltpu.sync_copy(x_hbm.at[pl.ds(tbase, COPY_TAIL)],
                        copy_v.at[pl.ds(0, COPY_TAIL)])
        pltpu.sync_copy(copy_v.at[pl.ds(0, COPY_TAIL)],
                        out_hbm.at[pl.ds(tbase, COPY_TAIL)])

    pair_base = wid * PAIRS_PER_W

    def chunk_body(k, carry):
        pbase = pair_base + k * C
        pltpu.sync_copy(idx_hbm.at[pl.ds(2 * pbase, 2 * C)], idx_v)
        pltpu.async_copy(x_hbm.at[idx_v], rows_v, sem).wait()

        def pair_body(p, c2):
            for v in range(NVEC):
                a = rows_v[2 * p, pl.ds(16 * v, 16)]
                b = rows_v[2 * p + 1, pl.ds(16 * v, 16)]
                acc_v[p, pl.ds(16 * v, 16)] = (a + b) * 0.5
            return c2

        lax.fori_loop(0, C, pair_body, 0)
        pltpu.sync_copy(acc_v, out_hbm.at[pl.ds(N_NODES + pbase, C)])
        return carry

    lax.fori_loop(0, NCHUNK, chunk_body, 0)


def kernel(inputs, pool_idx):
    idx_flat = pool_idx.reshape(-1)
    return _pool_kernel(inputs, idx_flat)


# speedup vs baseline: 3.6591x; 1.4490x over previous
import functools

import jax
import jax.numpy as jnp
from jax import lax
from jax.experimental import pallas as pl
from jax.experimental.pallas import tpu as pltpu
from jax.experimental.pallas import tpu_sc as plsc

N_NODES = 10000
N_PAIRS = 160000
D_FEAT = 256

NC = 2
NS = 16
NW = NC * NS

PAIRS_PER_W = N_PAIRS // NW
C = 40
NCHUNK = PAIRS_PER_W // C
NB = 4
NA = 2
NSTEADY = 30
NVEC = D_FEAT // 16

ROWS_PER_W = (N_NODES // (8 * NW)) * 8
COPY_TAIL = N_NODES - ROWS_PER_W * NW


@functools.partial(
    pl.kernel,
    mesh=plsc.VectorSubcoreMesh(core_axis_name="c", subcore_axis_name="s"),
    out_type=jax.ShapeDtypeStruct((N_NODES + N_PAIRS, D_FEAT), jnp.float32),
    scratch_types=[
        pltpu.VMEM((2 * PAIRS_PER_W,), jnp.int32),
        pltpu.VMEM((NB, 2 * C, D_FEAT), jnp.float32),
        pltpu.VMEM((NA, C, D_FEAT), jnp.float32),
    ] + [pltpu.SemaphoreType.DMA] * (NB + NA),
)
def _pool_kernel(x_hbm, idx_hbm, out_hbm, idx_v, rows_v, acc_v,
                 g0, g1, g2, g3, s0, s1):
    gsem = (g0, g1, g2, g3)
    ssem = (s0, s1)
    wid = lax.axis_index("s") * NC + lax.axis_index("c")
    pair_base = wid * PAIRS_PER_W

    pltpu.sync_copy(idx_hbm.at[pl.ds(2 * pair_base, 2 * PAIRS_PER_W)], idx_v)

    def start_gather(k, b):
        pltpu.async_copy(
            x_hbm.at[idx_v.at[pl.ds(k * (2 * C), 2 * C)]],
            rows_v.at[b], gsem[b])

    def wait_gather(b):
        pltpu.make_async_copy(x_hbm.at[idx_v.at[pl.ds(0, 2 * C)]],
                              rows_v.at[b], gsem[b]).wait()

    def start_store(k, a):
        pltpu.async_copy(acc_v.at[a],
                         out_hbm.at[pl.ds(N_NODES + pair_base + k * C, C)],
                         ssem[a])

    def wait_store(a):
        pltpu.make_async_copy(acc_v.at[a],
                              out_hbm.at[pl.ds(N_NODES, C)], ssem[a]).wait()

    for b in range(NB):
        start_gather(b, b)

    rbase = wid * ROWS_PER_W
    for t in range(7):
        pltpu.sync_copy(x_hbm.at[pl.ds(rbase + t * C, C)], acc_v.at[t % NA])
        pltpu.sync_copy(acc_v.at[t % NA], out_hbm.at[pl.ds(rbase + t * C, C)])
    pltpu.sync_copy(x_hbm.at[pl.ds(rbase + 280, 32)],
                    acc_v.at[1].at[pl.ds(0, 32)])
    pltpu.sync_copy(acc_v.at[1].at[pl.ds(0, 32)],
                    out_hbm.at[pl.ds(rbase + 280, 32)])

    @pl.when(wid == 0)
    def _copy_tail():
        tbase = NW * ROWS_PER_W
        pltpu.sync_copy(x_hbm.at[pl.ds(tbase, COPY_TAIL)],
                        acc_v.at[0].at[pl.ds(0, COPY_TAIL)])
        pltpu.sync_copy(acc_v.at[0].at[pl.ds(0, COPY_TAIL)],
                        out_hbm.at[pl.ds(tbase, COPY_TAIL)])

    def compute_chunk(b, a):
        def pair_body(p, c2):
            for v in range(NVEC):
                x = rows_v[b, 2 * p, pl.ds(16 * v, 16)]
                y = rows_v[b, 2 * p + 1, pl.ds(16 * v, 16)]
                acc_v[a, p, pl.ds(16 * v, 16)] = (x + y) * 0.5
            return c2
        lax.fori_loop(0, C, pair_body, 0)

    def group_body(kk, carry):
        for b in range(NB):
            k = kk * NB + b
            a = b % NA
            wait_gather(b)

            if b >= NA:
                wait_store(a)
            else:
                @pl.when(kk > 0)
                def _():
                    wait_store(a)

            compute_chunk(b, a)
            start_store(k, a)
            start_gather(k + NB, b)
        return carry

    lax.fori_loop(0, NSTEADY, group_body, 0)

    for k in range(NSTEADY * NB, NCHUNK):
        b = k % NB
        a = k % NA
        wait_gather(b)
        wait_store(a)
        compute_chunk(b, a)
        start_store(k, a)
        if k + NB < NCHUNK:
            start_gather(k + NB, b)
    for a in range(NA):
        wait_store(a)


def kernel(inputs, pool_idx):
    idx_flat = pool_idx.reshape(-1)
    return _pool_kernel(inputs, idx_flat)


# speedup vs baseline: 10.5399x; 2.8804x over previous
import functools

import jax
import jax.numpy as jnp
from jax import lax
from jax.experimental import pallas as pl
from jax.experimental.pallas import tpu as pltpu
from jax.experimental.pallas import tpu_sc as plsc

N_NODES = 10000
N_PAIRS = 160000
D_FEAT = 256

NC = 2
NS = 16
NW = NC * NS

PAIRS_PER_W = N_PAIRS // NW
C = 40
NCHUNK = PAIRS_PER_W // C
NB = 4
NA = 2
NSTEADY = 30
NVEC = D_FEAT // 16

ROWS_PER_W = (N_NODES // (8 * NW)) * 8
COPY_TAIL = N_NODES - ROWS_PER_W * NW


@functools.partial(
    pl.kernel,
    mesh=plsc.VectorSubcoreMesh(core_axis_name="c", subcore_axis_name="s"),
    out_type=jax.ShapeDtypeStruct((N_NODES + N_PAIRS, D_FEAT), jnp.float32),
    scratch_types=[
        pltpu.VMEM((PAIRS_PER_W,), jnp.int32),
        pltpu.VMEM((PAIRS_PER_W,), jnp.int32),
        pltpu.VMEM((NB, C, D_FEAT), jnp.float32),
        pltpu.VMEM((NB, C, D_FEAT), jnp.float32),
        pltpu.VMEM((NA, C, D_FEAT), jnp.float32),
    ] + [pltpu.SemaphoreType.DMA] * (NB + NA),
)
def _pool_kernel(x_hbm, idxl_hbm, idxr_hbm, out_hbm,
                 idxl_v, idxr_v, rl_v, rr_v, acc_v,
                 g0, g1, g2, g3, s0, s1):
    gsem = (g0, g1, g2, g3)
    ssem = (s0, s1)
    wid = lax.axis_index("s") * NC + lax.axis_index("c")
    pair_base = wid * PAIRS_PER_W

    pltpu.sync_copy(idxl_hbm.at[pl.ds(pair_base, PAIRS_PER_W)], idxl_v)
    pltpu.sync_copy(idxr_hbm.at[pl.ds(pair_base, PAIRS_PER_W)], idxr_v)

    def start_gather(k, b):
        pltpu.async_copy(x_hbm.at[idxl_v.at[pl.ds(k * C, C)]],
                         rl_v.at[b], gsem[b])
        pltpu.async_copy(x_hbm.at[idxr_v.at[pl.ds(k * C, C)]],
                         rr_v.at[b], gsem[b])

    def wait_gather(b):
        pltpu.make_async_copy(x_hbm.at[idxl_v.at[pl.ds(0, C)]],
                              rl_v.at[b], gsem[b]).wait()
        pltpu.make_async_copy(x_hbm.at[idxr_v.at[pl.ds(0, C)]],
                              rr_v.at[b], gsem[b]).wait()

    def start_store(k, a):
        pltpu.async_copy(acc_v.at[a],
                         out_hbm.at[pl.ds(N_NODES + pair_base + k * C, C)],
                         ssem[a])

    def wait_store(a):
        pltpu.make_async_copy(acc_v.at[a],
                              out_hbm.at[pl.ds(N_NODES, C)], ssem[a]).wait()

    for b in range(NB):
        start_gather(b, b)

    rbase = wid * ROWS_PER_W
    for t in range(7):
        pltpu.sync_copy(x_hbm.at[pl.ds(rbase + t * C, C)], acc_v.at[t % NA])
        pltpu.sync_copy(acc_v.at[t % NA], out_hbm.at[pl.ds(rbase + t * C, C)])
    pltpu.sync_copy(x_hbm.at[pl.ds(rbase + 280, 32)],
                    acc_v.at[1].at[pl.ds(0, 32)])
    pltpu.sync_copy(acc_v.at[1].at[pl.ds(0, 32)],
                    out_hbm.at[pl.ds(rbase + 280, 32)])

    @pl.when(wid == 0)
    def _copy_tail():
        tbase = NW * ROWS_PER_W
        pltpu.sync_copy(x_hbm.at[pl.ds(tbase, COPY_TAIL)],
                        acc_v.at[0].at[pl.ds(0, COPY_TAIL)])
        pltpu.sync_copy(acc_v.at[0].at[pl.ds(0, COPY_TAIL)],
                        out_hbm.at[pl.ds(tbase, COPY_TAIL)])

    def compute_chunk(b, a):
        @plsc.parallel_loop(0, C, unroll=4)
        def _pair(p):
            for v in range(NVEC):
                xl = rl_v[b, p, pl.ds(16 * v, 16)]
                xr = rr_v[b, p, pl.ds(16 * v, 16)]
                acc_v[a, p, pl.ds(16 * v, 16)] = (xl + xr) * 0.5

    def group_body(kk, carry):
        for b in range(NB):
            k = kk * NB + b
            a = b % NA
            wait_gather(b)

            if b >= NA:
                wait_store(a)
            else:
                @pl.when(kk > 0)
                def _():
                    wait_store(a)

            compute_chunk(b, a)
            start_store(k, a)
            start_gather(k + NB, b)
        return carry

    lax.fori_loop(0, NSTEADY, group_body, 0)

    for k in range(NSTEADY * NB, NCHUNK):
        b = k % NB
        a = k % NA
        wait_gather(b)
        wait_store(a)
        compute_chunk(b, a)
        start_store(k, a)
        if k + NB < NCHUNK:
            start_gather(k + NB, b)
    for a in range(NA):
        wait_store(a)


def kernel(inputs, pool_idx):
    idx_l = pool_idx[:, 0]
    idx_r = pool_idx[:, 1]
    return _pool_kernel(inputs, idx_l, idx_r)
